# baseline (device time: 149203 ns/iter reference)
import jax
import jax.numpy as jnp
import numpy as np
from jax import lax
from jax.experimental import pallas as pl
from jax.experimental.pallas import tpu as pltpu

N_DEV = 8
B, SQ, D, DH, HL = 2, 128, 512, 64, 4
HD = HL * DH


def _rope_consts():
    inv = 1.0 / (10000.0 ** (np.arange(0, DH, 2) / DH))
    pos = np.arange(SQ)[:, None] * inv[None, :]
    cos = np.repeat(np.cos(pos), 2, axis=-1)
    sin = np.repeat(np.sin(pos), 2, axis=-1)
    cos_t = np.tile(cos, (1, HL)).astype(np.float32)
    sin_t = np.tile(sin, (1, HL))
    even = (np.arange(HD) % 2 == 0)[None, :]
    sin_e = np.where(even, -sin_t, 0.0).astype(np.float32)
    sin_o = np.where(~even, sin_t, 0.0).astype(np.float32)
    return cos_t, sin_e, sin_o


def kernel(x, Wq, Wk, Wv, Wo):
    cos_t, sin_e, sin_o = _rope_consts()

    def body(x_ref, wq_ref, wk_ref, wv_ref, wo_ref, cos_ref, sine_ref, sino_ref,
             out_ref, pkt_ref, send_sems, recv_sems):
        my = lax.axis_index("i")
        left = lax.rem(my + N_DEV - 1, N_DEV)
        right = lax.rem(my + 1, N_DEV)

        barrier = pltpu.get_barrier_semaphore()
        for nbr in (left, right):
            pl.semaphore_signal(barrier, inc=1, device_id=(nbr,),
                                device_id_type=pl.DeviceIdType.MESH)
        pl.semaphore_wait(barrier, 2)

        def rope(t):
            tm = jnp.concatenate([t[:, 1:], t[:, :1]], axis=1)
            tp = jnp.concatenate([t[:, -1:], t[:, :-1]], axis=1)
            return t * cos_ref[:, :] + tm * sine_ref[:, :] + tp * sino_ref[:, :]

        def add_contribution(slot, is_init):
            wq = wq_ref[:, :]
            wk = wk_ref[:, :]
            wv = wv_ref[:, :]
            for b in range(B):
                xf = pkt_ref[slot, b]
                q = rope(jnp.dot(xf, wq, preferred_element_type=jnp.float32))
                k = rope(jnp.dot(xf, wk, preferred_element_type=jnp.float32))
                v = jnp.dot(xf, wv, preferred_element_type=jnp.float32)
                y = None
                for hh in range(HL):
                    sl = slice(hh * DH, (hh + 1) * DH)
                    s = lax.dot_general(
                        q[:, sl], k[:, sl], (((1,), (1,)), ((), ())),
                        preferred_element_type=jnp.float32) * 0.125
                    m = jnp.max(s, axis=1, keepdims=True)
                    e = jnp.exp(s - m)
                    w = e / jnp.sum(e, axis=1, keepdims=True)
                    ctx = jnp.dot(w, v[:, sl], preferred_element_type=jnp.float32)
                    part = jnp.dot(ctx, wo_ref[sl, :], preferred_element_type=jnp.float32)
                    y = part if y is None else y + part
                if is_init:
                    pkt_ref[slot, B + b] = y
                else:
                    pkt_ref[slot, B + b] = pkt_ref[slot, B + b] + y

        for b in range(B):
            pkt_ref[0, b] = x_ref[b]
        add_contribution(0, True)

        def hop(h, carry):
            rdma = pltpu.make_async_remote_copy(
                src_ref=pkt_ref.at[h - 1],
                dst_ref=pkt_ref.at[h],
                send_sem=send_sems.at[h - 1],
                recv_sem=recv_sems.at[h - 1],
                device_id=(right,),
                device_id_type=pl.DeviceIdType.MESH,
            )
            rdma.start()
            rdma.wait()
            add_contribution(h, False)
            return carry

        lax.fori_loop(1, N_DEV, hop, 0)

        rdma = pltpu.make_async_remote_copy(
            src_ref=pkt_ref.at[N_DEV - 1, pl.ds(B, B)],
            dst_ref=pkt_ref.at[0, pl.ds(0, B)],
            send_sem=send_sems.at[N_DEV - 1],
            recv_sem=recv_sems.at[N_DEV - 1],
            device_id=(right,),
            device_id_type=pl.DeviceIdType.MESH,
        )
        rdma.start()
        rdma.wait()
        out_ref[:, :, :] = pkt_ref[0, 0:B]

        def exit_barrier(sem):
            for nbr in (left, right):
                pl.semaphore_signal(sem, inc=1, device_id=(nbr,),
                                    device_id_type=pl.DeviceIdType.MESH)
            pl.semaphore_wait(sem, 2)

        pl.run_scoped(exit_barrier, pltpu.SemaphoreType.REGULAR)

    vmem = pl.BlockSpec(memory_space=pltpu.VMEM)
    return pl.pallas_call(
        body,
        out_shape=jax.ShapeDtypeStruct((B, SQ, D), jnp.float32),
        in_specs=[vmem] * 8,
        out_specs=vmem,
        scratch_shapes=[
            pltpu.VMEM((N_DEV, 2 * B, SQ, D), jnp.float32),
            pltpu.SemaphoreType.DMA((N_DEV,)),
            pltpu.SemaphoreType.DMA((N_DEV,)),
        ],
        compiler_params=pltpu.CompilerParams(collective_id=0),
    )(x, Wq, Wk, Wv, Wo, cos_t, sin_e, sin_o)


# device time: 61178 ns/iter; 2.4388x vs baseline; 2.4388x over previous
import jax
import jax.numpy as jnp
import numpy as np
from jax import lax
from jax.experimental import pallas as pl
from jax.experimental.pallas import tpu as pltpu

N_DEV = 8
B, SQ, D, DH, HL = 2, 128, 512, 64, 4
HD = HL * DH


def _rope_consts():
    inv = 1.0 / (10000.0 ** (np.arange(0, DH, 2) / DH))
    pos = np.arange(SQ)[:, None] * inv[None, :]
    cos = np.repeat(np.cos(pos), 2, axis=-1)
    sin = np.repeat(np.sin(pos), 2, axis=-1)
    cos_t = np.tile(cos, (1, HL)).astype(np.float32)
    sin_t = np.tile(sin, (1, HL))
    even = (np.arange(HD) % 2 == 0)[None, :]
    sin_e = np.where(even, -sin_t, 0.0).astype(np.float32)
    sin_o = np.where(~even, sin_t, 0.0).astype(np.float32)
    return cos_t, sin_e, sin_o


def kernel(x, Wq, Wk, Wv, Wo):
    cos_t, sin_e, sin_o = _rope_consts()
    bf = jnp.bfloat16

    def body(x_ref, wq_ref, wk_ref, wv_ref, wo_ref, cos_ref, sine_ref, sino_ref,
             out_ref, xbuf, accbuf, xs_send, xs_recv, ac_send, ac_recv):
        my = lax.axis_index("i")
        left = lax.rem(my + N_DEV - 1, N_DEV)
        right = lax.rem(my + 1, N_DEV)

        barrier = pltpu.get_barrier_semaphore()
        for nbr in (left, right):
            pl.semaphore_signal(barrier, inc=1, device_id=(nbr,),
                                device_id_type=pl.DeviceIdType.MESH)
        pl.semaphore_wait(barrier, 2)

        def x_rdma(h, src=None):
            return pltpu.make_async_remote_copy(
                src_ref=xbuf.at[h - 1] if src is None else src,
                dst_ref=xbuf.at[h],
                send_sem=xs_send.at[h - 1],
                recv_sem=xs_recv.at[h - 1],
                device_id=(right,),
                device_id_type=pl.DeviceIdType.MESH,
            )

        def acc_rdma(h):
            return pltpu.make_async_remote_copy(
                src_ref=accbuf.at[h - 1],
                dst_ref=accbuf.at[h],
                send_sem=ac_send.at[h - 1],
                recv_sem=ac_recv.at[h - 1],
                device_id=(right,),
                device_id_type=pl.DeviceIdType.MESH,
            )

        def rope(t):
            tm = jnp.concatenate([t[:, 1:], t[:, :1]], axis=1)
            tp = jnp.concatenate([t[:, -1:], t[:, :-1]], axis=1)
            return t * cos_ref[:, :] + tm * sine_ref[:, :] + tp * sino_ref[:, :]

        def contribution(xf):
            q = rope(jnp.dot(xf, wq_ref[:, :], preferred_element_type=jnp.float32))
            k = rope(jnp.dot(xf, wk_ref[:, :], preferred_element_type=jnp.float32))
            v = jnp.dot(xf, wv_ref[:, :], preferred_element_type=jnp.float32).astype(bf)
            q = q.astype(bf)
            k = k.astype(bf)
            y = None
            for hh in range(HL):
                sl = slice(hh * DH, (hh + 1) * DH)
                s = lax.dot_general(
                    q[:, sl], k[:, sl], (((1,), (1,)), ((), ())),
                    preferred_element_type=jnp.float32) * 0.125
                m = jnp.max(s, axis=1, keepdims=True)
                e = jnp.exp(s - m)
                w = (e / jnp.sum(e, axis=1, keepdims=True)).astype(bf)
                ctx = jnp.dot(w, v[:, sl], preferred_element_type=jnp.float32)
                part = jnp.dot(ctx.astype(bf), wo_ref[sl, :],
                               preferred_element_type=jnp.float32)
                y = part if y is None else y + part
            return y

        x_rdma(1, src=x_ref).start()
        for b in range(B):
            accbuf[0, b] = contribution(x_ref[b]).astype(bf)
        acc_rdma(1).start()

        def hop(h, carry):
            x_rdma(h).wait_recv()

            @pl.when(h < N_DEV - 1)
            def _():
                x_rdma(h + 1).start()

            ys = [contribution(xbuf[h, b]) for b in range(B)]
            acc_rdma(h).wait_recv()
            for b in range(B):
                accbuf[h, b] = (accbuf[h, b].astype(jnp.float32) + ys[b]).astype(bf)
            acc_rdma(h + 1).start()

            x_rdma(h).wait_send()
            acc_rdma(h).wait_send()
            return carry

        lax.fori_loop(1, N_DEV, hop, 0)

        acc_rdma(N_DEV).wait_recv()
        out_ref[:, :, :] = accbuf[N_DEV, 0:B].astype(jnp.float32)
        acc_rdma(N_DEV).wait_send()

        def exit_barrier(sem):
            for nbr in (left, right):
                pl.semaphore_signal(sem, inc=1, device_id=(nbr,),
                                    device_id_type=pl.DeviceIdType.MESH)
            pl.semaphore_wait(sem, 2)

        pl.run_scoped(exit_barrier, pltpu.SemaphoreType.REGULAR)

    vmem = pl.BlockSpec(memory_space=pltpu.VMEM)
    return pl.pallas_call(
        body,
        out_shape=jax.ShapeDtypeStruct((B, SQ, D), jnp.float32),
        in_specs=[vmem] * 8,
        out_specs=vmem,
        scratch_shapes=[
            pltpu.VMEM((N_DEV, B, SQ, D), bf),
            pltpu.VMEM((N_DEV + 1, B, SQ, D), bf),
            pltpu.SemaphoreType.DMA((N_DEV,)),
            pltpu.SemaphoreType.DMA((N_DEV,)),
            pltpu.SemaphoreType.DMA((N_DEV,)),
            pltpu.SemaphoreType.DMA((N_DEV,)),
        ],
        compiler_params=pltpu.CompilerParams(collective_id=0),
    )(x.astype(bf), Wq.astype(bf), Wk.astype(bf), Wv.astype(bf), Wo.astype(bf),
      cos_t, sin_e, sin_o)


# device time: 55094 ns/iter; 2.7082x vs baseline; 1.1104x over previous
import jax
import jax.numpy as jnp
import numpy as np
from jax import lax
from jax.experimental import pallas as pl
from jax.experimental.pallas import tpu as pltpu

N_DEV = 8
B, SQ, D, DH, HL = 2, 128, 512, 64, 4
HD = HL * DH


def _rope_consts():
    inv = 1.0 / (10000.0 ** (np.arange(0, DH, 2) / DH))
    pos = np.arange(SQ)[:, None] * inv[None, :]
    cos = np.repeat(np.cos(pos), 2, axis=-1)
    sin = np.repeat(np.sin(pos), 2, axis=-1)
    cos_t = np.tile(cos, (1, HL)).astype(np.float32)
    sin_t = np.tile(sin, (1, HL))
    even = (np.arange(HD) % 2 == 0)[None, :]
    sin_e = np.where(even, -sin_t, 0.0).astype(np.float32)
    sin_o = np.where(~even, sin_t, 0.0).astype(np.float32)
    return cos_t, sin_e, sin_o


def kernel(x, Wq, Wk, Wv, Wo):
    cos_t, sin_e, sin_o = _rope_consts()
    bf = jnp.bfloat16

    def body(x_ref, wq_ref, wk_ref, wv_ref, wo_ref, cos_ref, sine_ref, sino_ref,
             out_ref,
             xb_cw, xb_ccw, ab_cw, ab_ccw,
             xs_cw_s, xs_cw_r, xs_ccw_s, xs_ccw_r,
             ac_cw_s, ac_cw_r, ac_ccw_s, ac_ccw_r):
        my = lax.axis_index("i")
        left = lax.rem(my + N_DEV - 1, N_DEV)
        right = lax.rem(my + 1, N_DEV)

        barrier = pltpu.get_barrier_semaphore()
        for nbr in (left, right):
            pl.semaphore_signal(barrier, inc=1, device_id=(nbr,),
                                device_id_type=pl.DeviceIdType.MESH)
        pl.semaphore_wait(barrier, 2)

        def rdma(buf, send_sems, recv_sems, h, dst, src=None):
            return pltpu.make_async_remote_copy(
                src_ref=buf.at[h - 1] if src is None else src,
                dst_ref=buf.at[h],
                send_sem=send_sems.at[h - 1],
                recv_sem=recv_sems.at[h - 1],
                device_id=(dst,),
                device_id_type=pl.DeviceIdType.MESH,
            )

        def x_cw(h, src=None):
            return rdma(xb_cw, xs_cw_s, xs_cw_r, h, right, src)

        def x_ccw(h, src=None):
            return rdma(xb_ccw, xs_ccw_s, xs_ccw_r, h, left, src)

        def a_cw(h):
            return rdma(ab_cw, ac_cw_s, ac_cw_r, h, right)

        def a_ccw(h):
            return rdma(ab_ccw, ac_ccw_s, ac_ccw_r, h, left)

        def rope(t):
            tm = jnp.concatenate([t[:, 1:], t[:, :1]], axis=1)
            tp = jnp.concatenate([t[:, -1:], t[:, :-1]], axis=1)
            return t * cos_ref[:, :] + tm * sine_ref[:, :] + tp * sino_ref[:, :]

        def contribution(xf):
            q = rope(jnp.dot(xf, wq_ref[:, :], preferred_element_type=jnp.float32))
            k = rope(jnp.dot(xf, wk_ref[:, :], preferred_element_type=jnp.float32))
            v = jnp.dot(xf, wv_ref[:, :], preferred_element_type=jnp.float32).astype(bf)
            q = q.astype(bf)
            k = k.astype(bf)
            y = None
            for hh in range(HL):
                sl = slice(hh * DH, (hh + 1) * DH)
                s = lax.dot_general(
                    q[:, sl], k[:, sl], (((1,), (1,)), ((), ())),
                    preferred_element_type=jnp.float32) * 0.125
                m = jnp.max(s, axis=1, keepdims=True)
                e = jnp.exp(s - m)
                w = (e / jnp.sum(e, axis=1, keepdims=True)).astype(bf)
                ctx = jnp.dot(w, v[:, sl], preferred_element_type=jnp.float32)
                part = jnp.dot(ctx.astype(bf), wo_ref[sl, :],
                               preferred_element_type=jnp.float32)
                y = part if y is None else y + part
            return y

        x_cw(1, src=x_ref.at[0]).start()
        x_ccw(1, src=x_ref.at[1]).start()
        ab_cw[0] = contribution(x_ref[0]).astype(bf)
        ab_ccw[0] = contribution(x_ref[1]).astype(bf)
        a_cw(1).start()
        a_ccw(1).start()

        def hop(h, carry):
            x_cw(h).wait_recv()
            x_ccw(h).wait_recv()

            @pl.when(h < N_DEV - 1)
            def _():
                x_cw(h + 1).start()
                x_ccw(h + 1).start()

            y0 = contribution(xb_cw[h])
            a_cw(h).wait_recv()
            ab_cw[h] = (ab_cw[h].astype(jnp.float32) + y0).astype(bf)
            a_cw(h + 1).start()

            y1 = contribution(xb_ccw[h])
            a_ccw(h).wait_recv()
            ab_ccw[h] = (ab_ccw[h].astype(jnp.float32) + y1).astype(bf)
            a_ccw(h + 1).start()

            x_cw(h).wait_send()
            x_ccw(h).wait_send()
            a_cw(h).wait_send()
            a_ccw(h).wait_send()
            return carry

        lax.fori_loop(1, N_DEV, hop, 0)

        a_cw(N_DEV).wait_recv()
        out_ref[0] = ab_cw[N_DEV].astype(jnp.float32)
        a_ccw(N_DEV).wait_recv()
        out_ref[1] = ab_ccw[N_DEV].astype(jnp.float32)
        a_cw(N_DEV).wait_send()
        a_ccw(N_DEV).wait_send()

        def exit_barrier(sem):
            for nbr in (left, right):
                pl.semaphore_signal(sem, inc=1, device_id=(nbr,),
                                    device_id_type=pl.DeviceIdType.MESH)
            pl.semaphore_wait(sem, 2)

        pl.run_scoped(exit_barrier, pltpu.SemaphoreType.REGULAR)

    vmem = pl.BlockSpec(memory_space=pltpu.VMEM)
    return pl.pallas_call(
        body,
        out_shape=jax.ShapeDtypeStruct((B, SQ, D), jnp.float32),
        in_specs=[vmem] * 8,
        out_specs=vmem,
        scratch_shapes=[
            pltpu.VMEM((N_DEV, SQ, D), bf),
            pltpu.VMEM((N_DEV, SQ, D), bf),
            pltpu.VMEM((N_DEV + 1, SQ, D), bf),
            pltpu.VMEM((N_DEV + 1, SQ, D), bf),
            pltpu.SemaphoreType.DMA((N_DEV,)),
            pltpu.SemaphoreType.DMA((N_DEV,)),
            pltpu.SemaphoreType.DMA((N_DEV,)),
            pltpu.SemaphoreType.DMA((N_DEV,)),
            pltpu.SemaphoreType.DMA((N_DEV,)),
            pltpu.SemaphoreType.DMA((N_DEV,)),
            pltpu.SemaphoreType.DMA((N_DEV,)),
            pltpu.SemaphoreType.DMA((N_DEV,)),
        ],
        compiler_params=pltpu.CompilerParams(collective_id=0),
    )(x.astype(bf), Wq.astype(bf), Wk.astype(bf), Wv.astype(bf), Wo.astype(bf),
      cos_t, sin_e, sin_o)


# device time: 42935 ns/iter; 3.4751x vs baseline; 1.2832x over previous
import jax
import jax.numpy as jnp
import numpy as np
from jax import lax
from jax.experimental import pallas as pl
from jax.experimental.pallas import tpu as pltpu

N_DEV = 8
B, SQ, D, DH, HL = 2, 128, 512, 64, 4
HD = HL * DH
R = B * SQ


def _consts():
    inv = 1.0 / (10000.0 ** (np.arange(0, DH, 2) / DH))
    pos = np.arange(SQ)[:, None] * inv[None, :]
    cos = np.repeat(np.cos(pos), 2, axis=-1)
    sin = np.repeat(np.sin(pos), 2, axis=-1)
    cos_t = np.tile(cos, (B, HL)).astype(np.float32)
    sin_t = np.tile(sin, (B, HL))
    even = (np.arange(HD) % 2 == 0)[None, :]
    sin_e = np.where(even, -sin_t, 0.0).astype(np.float32)
    sin_o = np.where(~even, sin_t, 0.0).astype(np.float32)
    blk = np.arange(R) // SQ
    mask = np.where(blk[:, None] == blk[None, :], 0.0, -1e9).astype(np.float32)
    return cos_t, sin_e, sin_o, mask


def kernel(x, Wq, Wk, Wv, Wo):
    cos_t, sin_e, sin_o, mask = _consts()
    bf = jnp.bfloat16

    def body(x_ref, wq_ref, wk_ref, wv_ref, wo_ref, cos_ref, sine_ref, sino_ref,
             mask_ref, out_ref,
             xb, ab_cw, ab_ccw,
             xs_cw_s, xs_cw_r, xs_ccw_s, xs_ccw_r,
             ac_cw_s, ac_cw_r, ac_ccw_s, ac_ccw_r):
        my = lax.axis_index("i")
        left = lax.rem(my + N_DEV - 1, N_DEV)
        right = lax.rem(my + 1, N_DEV)

        barrier = pltpu.get_barrier_semaphore()
        for nbr in (left, right):
            pl.semaphore_signal(barrier, inc=1, device_id=(nbr,),
                                device_id_type=pl.DeviceIdType.MESH)
        pl.semaphore_wait(barrier, 2)

        def x_cw(h, src=None):
            return pltpu.make_async_remote_copy(
                src_ref=xb.at[h - 1, 0] if src is None else src,
                dst_ref=xb.at[h, 0],
                send_sem=xs_cw_s.at[h - 1], recv_sem=xs_cw_r.at[h - 1],
                device_id=(right,), device_id_type=pl.DeviceIdType.MESH)

        def x_ccw(h, src=None):
            return pltpu.make_async_remote_copy(
                src_ref=xb.at[h - 1, 1] if src is None else src,
                dst_ref=xb.at[h, 1],
                send_sem=xs_ccw_s.at[h - 1], recv_sem=xs_ccw_r.at[h - 1],
                device_id=(left,), device_id_type=pl.DeviceIdType.MESH)

        def a_cw(h):
            return pltpu.make_async_remote_copy(
                src_ref=ab_cw.at[h - 1], dst_ref=ab_cw.at[h],
                send_sem=ac_cw_s.at[h - 1], recv_sem=ac_cw_r.at[h - 1],
                device_id=(right,), device_id_type=pl.DeviceIdType.MESH)

        def a_ccw(h):
            return pltpu.make_async_remote_copy(
                src_ref=ab_ccw.at[h - 1], dst_ref=ab_ccw.at[h],
                send_sem=ac_ccw_s.at[h - 1], recv_sem=ac_ccw_r.at[h - 1],
                device_id=(left,), device_id_type=pl.DeviceIdType.MESH)

        def rope(t):
            tm = jnp.concatenate([t[:, 1:], t[:, :1]], axis=1)
            tp = jnp.concatenate([t[:, -1:], t[:, :-1]], axis=1)
            return t * cos_ref[:, :] + tm * sine_ref[:, :] + tp * sino_ref[:, :]

        def contribution(xf2):
            q = rope(jnp.dot(xf2, wq_ref[:, :], preferred_element_type=jnp.float32))
            k = rope(jnp.dot(xf2, wk_ref[:, :], preferred_element_type=jnp.float32))
            v = jnp.dot(xf2, wv_ref[:, :], preferred_element_type=jnp.float32).astype(bf)
            q = q.astype(bf)
            k = k.astype(bf)
            ctxs = []
            for hh in range(HL):
                sl = slice(hh * DH, (hh + 1) * DH)
                s = lax.dot_general(
                    q[:, sl], k[:, sl], (((1,), (1,)), ((), ())),
                    preferred_element_type=jnp.float32) * 0.125 + mask_ref[:, :]
                m = jnp.max(s, axis=1, keepdims=True)
                e = jnp.exp(s - m)
                w = (e / jnp.sum(e, axis=1, keepdims=True)).astype(bf)
                ctxs.append(
                    jnp.dot(w, v[:, sl], preferred_element_type=jnp.float32).astype(bf))
            ctx2 = jnp.concatenate(ctxs, axis=1)
            return jnp.dot(ctx2, wo_ref[:, :], preferred_element_type=jnp.float32)

        x_cw(1, src=x_ref.at[0]).start()
        x_ccw(1, src=x_ref.at[1]).start()
        y2 = contribution(x_ref[:, :, :].reshape(R, D))
        ab_cw[0] = y2[:SQ].astype(bf)
        ab_ccw[0] = y2[SQ:].astype(bf)
        a_cw(1).start()
        a_ccw(1).start()

        def hop(h, carry):
            x_cw(h).wait_recv()
            x_ccw(h).wait_recv()

            @pl.when(h < N_DEV - 1)
            def _():
                x_cw(h + 1).start()
                x_ccw(h + 1).start()

            y2 = contribution(xb[h].reshape(R, D))
            a_cw(h).wait_recv()
            ab_cw[h] = (ab_cw[h].astype(jnp.float32) + y2[:SQ]).astype(bf)
            a_cw(h + 1).start()
            a_ccw(h).wait_recv()
            ab_ccw[h] = (ab_ccw[h].astype(jnp.float32) + y2[SQ:]).astype(bf)
            a_ccw(h + 1).start()

            x_cw(h).wait_send()
            x_ccw(h).wait_send()
            a_cw(h).wait_send()
            a_ccw(h).wait_send()
            return carry

        lax.fori_loop(1, N_DEV, hop, 0)

        a_cw(N_DEV).wait_recv()
        out_ref[0] = ab_cw[N_DEV].astype(jnp.float32)
        a_ccw(N_DEV).wait_recv()
        out_ref[1] = ab_ccw[N_DEV].astype(jnp.float32)
        a_cw(N_DEV).wait_send()
        a_ccw(N_DEV).wait_send()

        def exit_barrier(sem):
            for nbr in (left, right):
                pl.semaphore_signal(sem, inc=1, device_id=(nbr,),
                                    device_id_type=pl.DeviceIdType.MESH)
            pl.semaphore_wait(sem, 2)

        pl.run_scoped(exit_barrier, pltpu.SemaphoreType.REGULAR)

    vmem = pl.BlockSpec(memory_space=pltpu.VMEM)
    return pl.pallas_call(
        body,
        out_shape=jax.ShapeDtypeStruct((B, SQ, D), jnp.float32),
        in_specs=[vmem] * 9,
        out_specs=vmem,
        scratch_shapes=[
            pltpu.VMEM((N_DEV, B, SQ, D), bf),
            pltpu.VMEM((N_DEV + 1, SQ, D), bf),
            pltpu.VMEM((N_DEV + 1, SQ, D), bf),
            pltpu.SemaphoreType.DMA((N_DEV,)),
            pltpu.SemaphoreType.DMA((N_DEV,)),
            pltpu.SemaphoreType.DMA((N_DEV,)),
            pltpu.SemaphoreType.DMA((N_DEV,)),
            pltpu.SemaphoreType.DMA((N_DEV,)),
            pltpu.SemaphoreType.DMA((N_DEV,)),
            pltpu.SemaphoreType.DMA((N_DEV,)),
            pltpu.SemaphoreType.DMA((N_DEV,)),
        ],
        compiler_params=pltpu.CompilerParams(collective_id=0),
    )(x.astype(bf), Wq.astype(bf), Wk.astype(bf), Wv.astype(bf), Wo.astype(bf),
      cos_t, sin_e, sin_o, mask)


# device time: 42006 ns/iter; 3.5519x vs baseline; 1.0221x over previous
import jax
import jax.numpy as jnp
import numpy as np
from jax import lax
from jax.experimental import pallas as pl
from jax.experimental.pallas import tpu as pltpu

N_DEV = 8
B, SQ, D, DH, HL = 2, 128, 512, 64, 4
HD = HL * DH
R = B * SQ


def _consts():
    inv = 1.0 / (10000.0 ** (np.arange(0, DH, 2) / DH))
    pos = np.arange(SQ)[:, None] * inv[None, :]
    cos = np.repeat(np.cos(pos), 2, axis=-1)
    sin = np.repeat(np.sin(pos), 2, axis=-1)
    cos_t = np.tile(cos, (B, HL)).astype(np.float32)
    sin_t = np.tile(sin, (B, HL))
    even = (np.arange(HD) % 2 == 0)[None, :]
    sin_e = np.where(even, -sin_t, 0.0).astype(np.float32)
    sin_o = np.where(~even, sin_t, 0.0).astype(np.float32)
    blk = np.arange(R) // SQ
    mask = np.where(blk[:, None] == blk[None, :], 0.0, -1e9).astype(np.float32)
    return cos_t, sin_e, sin_o, mask


def kernel(x, Wq, Wk, Wv, Wo):
    cos_t, sin_e, sin_o, mask = _consts()
    bf = jnp.bfloat16

    def body(x_ref, wq_ref, wk_ref, wv_ref, wo_ref, cos_ref, sine_ref, sino_ref,
             mask_ref, out_ref,
             xb, ab_cw, ab_ccw,
             xs_cw_s, xs_cw_r, xs_ccw_s, xs_ccw_r,
             ac_cw_s, ac_cw_r, ac_ccw_s, ac_ccw_r):
        my = lax.axis_index("i")
        left = lax.rem(my + N_DEV - 1, N_DEV)
        right = lax.rem(my + 1, N_DEV)

        barrier = pltpu.get_barrier_semaphore()
        for nbr in (left, right):
            pl.semaphore_signal(barrier, inc=1, device_id=(nbr,),
                                device_id_type=pl.DeviceIdType.MESH)
        pl.semaphore_wait(barrier, 2)

        def x_cw(h, src=None):
            return pltpu.make_async_remote_copy(
                src_ref=xb.at[h - 1, 0] if src is None else src,
                dst_ref=xb.at[h, 0],
                send_sem=xs_cw_s.at[h - 1], recv_sem=xs_cw_r.at[h - 1],
                device_id=(right,), device_id_type=pl.DeviceIdType.MESH)

        def x_ccw(h, src=None):
            return pltpu.make_async_remote_copy(
                src_ref=xb.at[h - 1, 1] if src is None else src,
                dst_ref=xb.at[h, 1],
                send_sem=xs_ccw_s.at[h - 1], recv_sem=xs_ccw_r.at[h - 1],
                device_id=(left,), device_id_type=pl.DeviceIdType.MESH)

        def a_cw(h):
            return pltpu.make_async_remote_copy(
                src_ref=ab_cw.at[h - 1], dst_ref=ab_cw.at[h],
                send_sem=ac_cw_s.at[h - 1], recv_sem=ac_cw_r.at[h - 1],
                device_id=(right,), device_id_type=pl.DeviceIdType.MESH)

        def a_ccw(h):
            return pltpu.make_async_remote_copy(
                src_ref=ab_ccw.at[h - 1], dst_ref=ab_ccw.at[h],
                send_sem=ac_ccw_s.at[h - 1], recv_sem=ac_ccw_r.at[h - 1],
                device_id=(left,), device_id_type=pl.DeviceIdType.MESH)

        def rope(t):
            tm = jnp.concatenate([t[:, 1:], t[:, :1]], axis=1)
            tp = jnp.concatenate([t[:, -1:], t[:, :-1]], axis=1)
            return t * cos_ref[:, :] + tm * sine_ref[:, :] + tp * sino_ref[:, :]

        def contribution(xf2):
            q = rope(jnp.dot(xf2, wq_ref[:, :],
                             preferred_element_type=jnp.float32).astype(bf))
            k = rope(jnp.dot(xf2, wk_ref[:, :],
                             preferred_element_type=jnp.float32).astype(bf))
            v = jnp.dot(xf2, wv_ref[:, :],
                        preferred_element_type=jnp.float32).astype(bf)
            ctxs = []
            for hh in range(HL):
                sl = slice(hh * DH, (hh + 1) * DH)
                s = lax.dot_general(
                    q[:, sl], k[:, sl], (((1,), (1,)), ((), ())),
                    preferred_element_type=jnp.float32) * 0.125 + mask_ref[:, :]
                e = jnp.exp(s)
                r = jnp.sum(e, axis=1, keepdims=True)
                ctx = jnp.dot(e.astype(bf), v[:, sl],
                              preferred_element_type=jnp.float32) / r
                ctxs.append(ctx.astype(bf))
            ctx2 = jnp.concatenate(ctxs, axis=1)
            return jnp.dot(ctx2, wo_ref[:, :], preferred_element_type=jnp.float32)

        x_cw(1, src=x_ref.at[0]).start()
        x_ccw(1, src=x_ref.at[1]).start()
        y2 = contribution(x_ref[:, :, :].reshape(R, D))
        ab_cw[0] = y2[:SQ].astype(bf)
        ab_ccw[0] = y2[SQ:].astype(bf)
        a_cw(1).start()
        a_ccw(1).start()

        def hop(h, carry):
            x_cw(h).wait_recv()
            x_ccw(h).wait_recv()

            @pl.when(h < N_DEV - 1)
            def _():
                x_cw(h + 1).start()
                x_ccw(h + 1).start()

            y2 = contribution(xb[h].reshape(R, D))
            a_cw(h).wait_recv()
            ab_cw[h] = (ab_cw[h].astype(jnp.float32) + y2[:SQ]).astype(bf)
            a_cw(h + 1).start()
            a_ccw(h).wait_recv()
            ab_ccw[h] = (ab_ccw[h].astype(jnp.float32) + y2[SQ:]).astype(bf)
            a_ccw(h + 1).start()

            x_cw(h).wait_send()
            x_ccw(h).wait_send()
            a_cw(h).wait_send()
            a_ccw(h).wait_send()
            return carry

        lax.fori_loop(1, N_DEV, hop, 0)

        a_cw(N_DEV).wait_recv()
        out_ref[0] = ab_cw[N_DEV].astype(jnp.float32)
        a_ccw(N_DEV).wait_recv()
        out_ref[1] = ab_ccw[N_DEV].astype(jnp.float32)
        a_cw(N_DEV).wait_send()
        a_ccw(N_DEV).wait_send()

        def exit_barrier(sem):
            for nbr in (left, right):
                pl.semaphore_signal(sem, inc=1, device_id=(nbr,),
                                    device_id_type=pl.DeviceIdType.MESH)
            pl.semaphore_wait(sem, 2)

        pl.run_scoped(exit_barrier, pltpu.SemaphoreType.REGULAR)

    vmem = pl.BlockSpec(memory_space=pltpu.VMEM)
    return pl.pallas_call(
        body,
        out_shape=jax.ShapeDtypeStruct((B, SQ, D), jnp.float32),
        in_specs=[vmem] * 9,
        out_specs=vmem,
        scratch_shapes=[
            pltpu.VMEM((N_DEV, B, SQ, D), bf),
            pltpu.VMEM((N_DEV + 1, SQ, D), bf),
            pltpu.VMEM((N_DEV + 1, SQ, D), bf),
            pltpu.SemaphoreType.DMA((N_DEV,)),
            pltpu.SemaphoreType.DMA((N_DEV,)),
            pltpu.SemaphoreType.DMA((N_DEV,)),
            pltpu.SemaphoreType.DMA((N_DEV,)),
            pltpu.SemaphoreType.DMA((N_DEV,)),
            pltpu.SemaphoreType.DMA((N_DEV,)),
            pltpu.SemaphoreType.DMA((N_DEV,)),
            pltpu.SemaphoreType.DMA((N_DEV,)),
        ],
        compiler_params=pltpu.CompilerParams(collective_id=0),
    )(x.astype(bf), Wq.astype(bf), Wk.astype(bf), Wv.astype(bf), Wo.astype(bf),
      cos_t.astype(bf), sin_e.astype(bf), sin_o.astype(bf), mask)


# device time: 39110 ns/iter; 3.8150x vs baseline; 1.0740x over previous
import jax
import jax.numpy as jnp
import numpy as np
from jax import lax
from jax.experimental import pallas as pl
from jax.experimental.pallas import tpu as pltpu

N_DEV = 8
B, SQ, D, DH, HL = 2, 128, 512, 64, 4
HD = HL * DH
R = B * SQ


def _consts():
    inv = 1.0 / (10000.0 ** (np.arange(0, DH, 2) / DH))
    pos = np.arange(SQ)[:, None] * inv[None, :]
    cos = np.repeat(np.cos(pos), 2, axis=-1)
    sin = np.repeat(np.sin(pos), 2, axis=-1)
    cos_t = np.tile(cos, (B, HL)).astype(np.float32)
    sin_t = np.tile(sin, (B, HL))
    even = (np.arange(HD) % 2 == 0)[None, :]
    sin_e = np.where(even, -sin_t, 0.0).astype(np.float32)
    sin_o = np.where(~even, sin_t, 0.0).astype(np.float32)
    blk = np.arange(R) // SQ
    mask = np.where(blk[:, None] == blk[None, :], 0.0, -1e9).astype(np.float32)
    return cos_t, sin_e, sin_o, mask


_CYCLE = [0, 1, 2, 3, 7, 6, 5, 4]
_NEXT = np.zeros(N_DEV, np.int32)
_PREV = np.zeros(N_DEV, np.int32)
for _i, _p in enumerate(_CYCLE):
    _NEXT[_p] = _CYCLE[(_i + 1) % N_DEV]
    _PREV[_p] = _CYCLE[(_i - 1) % N_DEV]


def kernel(x, Wq, Wk, Wv, Wo):
    cos_t, sin_e, sin_o, mask = _consts()
    bf = jnp.bfloat16

    def body(x_ref, wq_ref, wk_ref, wv_ref, wo_ref, cos_ref, sine_ref, sino_ref,
             mask_ref, out_ref,
             xb, ab_cw, ab_ccw,
             xs_cw_s, xs_cw_r, xs_ccw_s, xs_ccw_r,
             ac_cw_s, ac_cw_r, ac_ccw_s, ac_ccw_r):
        my = lax.axis_index("i")

        def lookup(table):
            r = jnp.int32(table[0])
            for i in range(1, N_DEV):
                r = jnp.where(my == i, jnp.int32(table[i]), r)
            return r

        right = lookup(_NEXT)
        left = lookup(_PREV)

        barrier = pltpu.get_barrier_semaphore()
        for nbr in (left, right):
            pl.semaphore_signal(barrier, inc=1, device_id=(nbr,),
                                device_id_type=pl.DeviceIdType.MESH)
        pl.semaphore_wait(barrier, 2)

        def x_cw(h, src=None):
            return pltpu.make_async_remote_copy(
                src_ref=xb.at[h - 1, 0] if src is None else src,
                dst_ref=xb.at[h, 0],
                send_sem=xs_cw_s.at[h - 1], recv_sem=xs_cw_r.at[h - 1],
                device_id=(right,), device_id_type=pl.DeviceIdType.MESH)

        def x_ccw(h, src=None):
            return pltpu.make_async_remote_copy(
                src_ref=xb.at[h - 1, 1] if src is None else src,
                dst_ref=xb.at[h, 1],
                send_sem=xs_ccw_s.at[h - 1], recv_sem=xs_ccw_r.at[h - 1],
                device_id=(left,), device_id_type=pl.DeviceIdType.MESH)

        def a_cw(h):
            return pltpu.make_async_remote_copy(
                src_ref=ab_cw.at[h - 1], dst_ref=ab_cw.at[h],
                send_sem=ac_cw_s.at[h - 1], recv_sem=ac_cw_r.at[h - 1],
                device_id=(right,), device_id_type=pl.DeviceIdType.MESH)

        def a_ccw(h):
            return pltpu.make_async_remote_copy(
                src_ref=ab_ccw.at[h - 1], dst_ref=ab_ccw.at[h],
                send_sem=ac_ccw_s.at[h - 1], recv_sem=ac_ccw_r.at[h - 1],
                device_id=(left,), device_id_type=pl.DeviceIdType.MESH)

        def rope(t):
            tm = jnp.concatenate([t[:, 1:], t[:, :1]], axis=1)
            tp = jnp.concatenate([t[:, -1:], t[:, :-1]], axis=1)
            return t * cos_ref[:, :] + tm * sine_ref[:, :] + tp * sino_ref[:, :]

        def contribution(xf2):
            q = rope(jnp.dot(xf2, wq_ref[:, :],
                             preferred_element_type=jnp.float32).astype(bf))
            k = rope(jnp.dot(xf2, wk_ref[:, :],
                             preferred_element_type=jnp.float32).astype(bf))
            v = jnp.dot(xf2, wv_ref[:, :],
                        preferred_element_type=jnp.float32).astype(bf)
            ctxs = []
            for hh in range(HL):
                sl = slice(hh * DH, (hh + 1) * DH)
                s = lax.dot_general(
                    q[:, sl], k[:, sl], (((1,), (1,)), ((), ())),
                    preferred_element_type=jnp.float32) * 0.125 + mask_ref[:, :]
                e = jnp.exp(s)
                r = jnp.sum(e, axis=1, keepdims=True)
                ctx = jnp.dot(e.astype(bf), v[:, sl],
                              preferred_element_type=jnp.float32) / r
                ctxs.append(ctx.astype(bf))
            return jnp.concatenate(ctxs, axis=1)

        x_cw(1, src=x_ref.at[0]).start()
        x_ccw(1, src=x_ref.at[1]).start()
        ctx2 = contribution(x_ref[:, :, :].reshape(R, D))
        ab_cw[0] = jnp.dot(ctx2[:SQ], wo_ref[:, :],
                           preferred_element_type=jnp.float32).astype(bf)
        a_cw(1).start()
        ab_ccw[0] = jnp.dot(ctx2[SQ:], wo_ref[:, :],
                            preferred_element_type=jnp.float32).astype(bf)
        a_ccw(1).start()

        def hop(h, carry):
            x_cw(h).wait_recv()
            x_ccw(h).wait_recv()

            @pl.when(h < N_DEV - 1)
            def _():
                x_cw(h + 1).start()
                x_ccw(h + 1).start()

            ctx2 = contribution(xb[h].reshape(R, D))

            def fold_cw():
                y = jnp.dot(ctx2[:SQ], wo_ref[:, :],
                            preferred_element_type=jnp.float32)
                a_cw(h).wait_recv()
                ab_cw[h] = (ab_cw[h].astype(jnp.float32) + y).astype(bf)
                a_cw(h + 1).start()

            def fold_ccw():
                y = jnp.dot(ctx2[SQ:], wo_ref[:, :],
                            preferred_element_type=jnp.float32)
                a_ccw(h).wait_recv()
                ab_ccw[h] = (ab_ccw[h].astype(jnp.float32) + y).astype(bf)
                a_ccw(h + 1).start()

            @pl.when(h % 2 == 1)
            def _():
                fold_cw()
                fold_ccw()

            @pl.when(h % 2 == 0)
            def _():
                fold_ccw()
                fold_cw()

            x_cw(h).wait_send()
            x_ccw(h).wait_send()
            a_cw(h).wait_send()
            a_ccw(h).wait_send()
            return carry

        lax.fori_loop(1, N_DEV, hop, 0)

        a_cw(N_DEV).wait_recv()
        out_ref[0] = ab_cw[N_DEV].astype(jnp.float32)
        a_ccw(N_DEV).wait_recv()
        out_ref[1] = ab_ccw[N_DEV].astype(jnp.float32)
        a_cw(N_DEV).wait_send()
        a_ccw(N_DEV).wait_send()

        def exit_barrier(sem):
            for nbr in (left, right):
                pl.semaphore_signal(sem, inc=1, device_id=(nbr,),
                                    device_id_type=pl.DeviceIdType.MESH)
            pl.semaphore_wait(sem, 2)

        pl.run_scoped(exit_barrier, pltpu.SemaphoreType.REGULAR)

    vmem = pl.BlockSpec(memory_space=pltpu.VMEM)
    return pl.pallas_call(
        body,
        out_shape=jax.ShapeDtypeStruct((B, SQ, D), jnp.float32),
        in_specs=[vmem] * 9,
        out_specs=vmem,
        scratch_shapes=[
            pltpu.VMEM((N_DEV, B, SQ, D), bf),
            pltpu.VMEM((N_DEV + 1, SQ, D), bf),
            pltpu.VMEM((N_DEV + 1, SQ, D), bf),
            pltpu.SemaphoreType.DMA((N_DEV,)),
            pltpu.SemaphoreType.DMA((N_DEV,)),
            pltpu.SemaphoreType.DMA((N_DEV,)),
            pltpu.SemaphoreType.DMA((N_DEV,)),
            pltpu.SemaphoreType.DMA((N_DEV,)),
            pltpu.SemaphoreType.DMA((N_DEV,)),
            pltpu.SemaphoreType.DMA((N_DEV,)),
            pltpu.SemaphoreType.DMA((N_DEV,)),
        ],
        compiler_params=pltpu.CompilerParams(collective_id=0),
    )(x.astype(bf), Wq.astype(bf), Wk.astype(bf), Wv.astype(bf), Wo.astype(bf),
      cos_t.astype(bf), sin_e.astype(bf), sin_o.astype(bf), mask)
